# baseline (speedup 1.0000x reference)
import functools

import jax
import jax.numpy as jnp
from jax import lax
from jax.experimental import pallas as pl
from jax.experimental.pallas import tpu as pltpu
from jax.experimental.pallas import tpu_sc as plsc

NC = 2
NS = 16
NW = NC * NS
K = 80
R = 1000


def _deg_body(dst_hbm, out_hbm, deg_sh, dst_v, ones_v, zero_v):
  c = lax.axis_index("c")
  s = lax.axis_index("s")
  wid = c * NS + s
  np_, = deg_sh.shape
  zt = np_ // NS
  nchunk, kd = dst_v.shape

  def fill_zero(j, _):
    zero_v[pl.ds(j * 16, 16)] = jnp.zeros((16,), jnp.float32)
    return 0
  lax.fori_loop(0, zt // 16, fill_zero, 0)

  def fill_one(j, _):
    ones_v[pl.ds(j * 16, 16)] = jnp.ones((16,), jnp.float32)
    return 0
  lax.fori_loop(0, kd // 16, fill_one, 0)

  pltpu.sync_copy(zero_v, deg_sh.at[pl.ds(s * zt, zt)])
  pltpu.sync_copy(dst_hbm.at[wid], dst_v)
  plsc.subcore_barrier()

  def body(j, _):
    pltpu.sync_copy(ones_v, deg_sh.at[dst_v.at[j]], add=True)
    return 0
  lax.fori_loop(0, nchunk, body, 0)

  plsc.subcore_barrier()
  pltpu.sync_copy(deg_sh.at[pl.ds(s * zt, zt)],
                  out_hbm.at[pl.ds(c * np_ + s * zt, zt)])


def _edge_body(hs_hbm, src_hbm, dst_hbm, out_hbm, acc_sh, src_v, dst_v,
               rows_v, sem, nchunk):
  c = lax.axis_index("c")
  s = lax.axis_index("s")
  wid = c * NS + s
  n = acc_sh.shape[0]
  rt = (n // NS + 7) // 8 * 8
  rt_last = n - (NS - 1) * rt
  ew = nchunk * K
  base = wid * ew

  @pl.when(s < NS - 1)
  def _():
    pltpu.sync_copy(hs_hbm.at[pl.ds(s * rt, rt)], acc_sh.at[pl.ds(s * rt, rt)])

  @pl.when(s == NS - 1)
  def _():
    pltpu.sync_copy(hs_hbm.at[pl.ds((NS - 1) * rt, rt_last)],
                    acc_sh.at[pl.ds((NS - 1) * rt, rt_last)])

  plsc.subcore_barrier()

  sem_g, sem_s = sem
  pltpu.sync_copy(src_hbm.at[pl.ds(base, K)], src_v.at[0])
  pltpu.sync_copy(dst_hbm.at[pl.ds(base, K)], dst_v.at[0])
  pltpu.async_copy(hs_hbm.at[src_v.at[0]], rows_v.at[0], sem_g)

  def body(j, _):
    b = lax.rem(j, 2)
    nb = 1 - b
    pltpu.make_async_copy(hs_hbm.at[src_v.at[b]], rows_v.at[b], sem_g).wait()

    @pl.when(j >= 1)
    def _():
      pltpu.make_async_copy(rows_v.at[nb], acc_sh.at[dst_v.at[nb]],
                            sem_s).wait()

    @pl.when(j + 1 < nchunk)
    def _():
      pltpu.sync_copy(src_hbm.at[pl.ds(base + (j + 1) * K, K)], src_v.at[nb])
      pltpu.sync_copy(dst_hbm.at[pl.ds(base + (j + 1) * K, K)], dst_v.at[nb])
      pltpu.async_copy(hs_hbm.at[src_v.at[nb]], rows_v.at[nb], sem_g)

    pltpu.async_copy(rows_v.at[b], acc_sh.at[dst_v.at[b]], sem_s, add=True)
    return 0
  lax.fori_loop(0, nchunk, body, 0)

  lb = (nchunk - 1) % 2
  pltpu.make_async_copy(rows_v.at[lb], acc_sh.at[dst_v.at[lb]], sem_s).wait()
  plsc.subcore_barrier()

  @pl.when(s < NS - 1)
  def _():
    pltpu.sync_copy(acc_sh.at[pl.ds(s * rt, rt)],
                    out_hbm.at[pl.ds(c * n + s * rt, rt)])

  @pl.when(s == NS - 1)
  def _():
    pltpu.sync_copy(acc_sh.at[pl.ds((NS - 1) * rt, rt_last)],
                    out_hbm.at[pl.ds(c * n + (NS - 1) * rt, rt_last)])


def _first_tc(x_ref, w_ref, degt_ref, hs_ref, dis_ref):
  d = jnp.sum(degt_ref[...], axis=1, keepdims=True) + 1.0
  dis = lax.rsqrt(d)
  xw = jnp.dot(x_ref[...], w_ref[...], preferred_element_type=jnp.float32)
  hs_ref[...] = xw * dis
  dis_ref[...] = dis


def _mid_tc(acc0_ref, acc1_ref, hs_ref, dis_ref, b_ref, w_ref, out_ref):
  dis = dis_ref[...]
  h = (acc0_ref[...] + acc1_ref[...] - hs_ref[...]) * dis + b_ref[...]
  h = jnp.maximum(h, 0.0)
  out_ref[...] = jnp.dot(h, w_ref[...],
                         preferred_element_type=jnp.float32) * dis


def _head_tc(acc0_ref, acc1_ref, hs_ref, dis_ref, b_ref, wfc_ref, bfc_ref,
             out_ref, colsum):
  i = pl.program_id(0)
  nblk = pl.num_programs(0)
  dis = dis_ref[...]
  h = (acc0_ref[...] + acc1_ref[...] - hs_ref[...]) * dis + b_ref[...]
  h = jnp.maximum(h, 0.0)

  @pl.when(i == 0)
  def _():
    colsum[...] = jnp.zeros_like(colsum)

  colsum[...] += jnp.sum(h, axis=0, keepdims=True)

  @pl.when(i == nblk - 1)
  def _():
    g = colsum[...] / (nblk * h.shape[0])
    out_ref[...] = jnp.dot(g, wfc_ref[...],
                           preferred_element_type=jnp.float32) + bfc_ref[...]


def kernel(x, edge_index, W1, b1, W2, b2, W3, b3, Wfc, bfc):
  n, d = x.shape
  h = W1.shape[1]
  o = Wfc.shape[1]
  e = edge_index.shape[1]
  ew = e // NW
  nchunk = ew // K
  np_ = ((n + 255) // 256) * 256
  nblk = n // R

  src_flat = edge_index[0]
  dst_flat = edge_index[1]
  kd = 80
  dst3b = edge_index[1].reshape(NW, ew // kd, kd)

  mesh = plsc.VectorSubcoreMesh(core_axis_name="c", subcore_axis_name="s")

  deg_kernel = pl.kernel(
      _deg_body,
      out_type=jax.ShapeDtypeStruct((NC * np_,), jnp.float32),
      mesh=mesh,
      scratch_types=[
          pltpu.VMEM_SHARED((np_,), jnp.float32),
          pltpu.VMEM((ew // kd, kd), jnp.int32),
          pltpu.VMEM((kd,), jnp.float32),
          pltpu.VMEM((np_ // NS,), jnp.float32),
      ],
  )
  deg_parts = deg_kernel(dst3b)
  degt = deg_parts.reshape(NC, np_).T[:n]

  edge_kernel = pl.kernel(
      functools.partial(_edge_body, nchunk=nchunk),
      out_type=jax.ShapeDtypeStruct((2 * n, h), jnp.float32),
      mesh=mesh,
      scratch_types=[
          pltpu.VMEM_SHARED((n, h), jnp.float32),
          pltpu.VMEM((2, K), jnp.int32),
          pltpu.VMEM((2, K), jnp.int32),
          pltpu.VMEM((2, K, h), jnp.float32),
          (pltpu.SemaphoreType.DMA, pltpu.SemaphoreType.DMA),
      ],
  )

  row = lambda i: (i, 0)
  row_hi = lambda i: (i + nblk, 0)
  fixed = lambda i: (0, 0)

  first = pl.pallas_call(
      _first_tc,
      grid=(nblk,),
      in_specs=[
          pl.BlockSpec((R, d), row),
          pl.BlockSpec((d, h), fixed),
          pl.BlockSpec((R, NC), row),
      ],
      out_specs=[
          pl.BlockSpec((R, h), row),
          pl.BlockSpec((R, 1), row),
      ],
      out_shape=[
          jax.ShapeDtypeStruct((n, h), jnp.float32),
          jax.ShapeDtypeStruct((n, 1), jnp.float32),
      ],
  )
  hs1, dis = first(x, W1, degt)

  def mid(acc, hs_prev, b, w):
    return pl.pallas_call(
        _mid_tc,
        grid=(nblk,),
        in_specs=[
            pl.BlockSpec((R, h), row),
            pl.BlockSpec((R, h), row_hi),
            pl.BlockSpec((R, h), row),
            pl.BlockSpec((R, 1), row),
            pl.BlockSpec((1, h), fixed),
            pl.BlockSpec((h, h), fixed),
        ],
        out_specs=pl.BlockSpec((R, h), row),
        out_shape=jax.ShapeDtypeStruct((n, h), jnp.float32),
    )(acc, acc, hs_prev, dis, b.reshape(1, h), w)

  acc1 = edge_kernel(hs1, src_flat, dst_flat)
  hs2 = mid(acc1, hs1, b1, W2)
  acc2 = edge_kernel(hs2, src_flat, dst_flat)
  hs3 = mid(acc2, hs2, b2, W3)
  acc3 = edge_kernel(hs3, src_flat, dst_flat)

  wfc_p = jnp.zeros((h, 128), jnp.float32).at[:, :o].set(Wfc)
  bfc_p = jnp.zeros((1, 128), jnp.float32).at[0, :o].set(bfc)

  head = pl.pallas_call(
      _head_tc,
      grid=(nblk,),
      in_specs=[
          pl.BlockSpec((R, h), row),
          pl.BlockSpec((R, h), row_hi),
          pl.BlockSpec((R, h), row),
          pl.BlockSpec((R, 1), row),
          pl.BlockSpec((1, h), fixed),
          pl.BlockSpec((h, 128), fixed),
          pl.BlockSpec((1, 128), fixed),
      ],
      out_specs=pl.BlockSpec((1, 128), fixed),
      out_shape=jax.ShapeDtypeStruct((1, 128), jnp.float32),
      scratch_shapes=[pltpu.VMEM((1, 128), jnp.float32)],
  )
  out = head(acc3, acc3, hs3, dis, b3.reshape(1, h), wfc_p, bfc_p)
  return out[0, :o]


# speedup vs baseline: 1.5386x; 1.5386x over previous
import functools

import jax
import jax.numpy as jnp
from jax import lax
from jax.experimental import pallas as pl
from jax.experimental.pallas import tpu as pltpu
from jax.experimental.pallas import tpu_sc as plsc

NC = 2
NS = 16
NW = NC * NS
K = 80
R = 1000


def _deg_body(dst_hbm, out_hbm, deg_sh, dst_v, ones_v, zero_v):
  c = lax.axis_index("c")
  s = lax.axis_index("s")
  wid = c * NS + s
  np_, = deg_sh.shape
  zt = np_ // NS
  nchunk, kd = dst_v.shape

  def fill_zero(j, _):
    zero_v[pl.ds(j * 16, 16)] = jnp.zeros((16,), jnp.float32)
    return 0
  lax.fori_loop(0, zt // 16, fill_zero, 0)

  def fill_one(j, _):
    ones_v[pl.ds(j * 16, 16)] = jnp.ones((16,), jnp.float32)
    return 0
  lax.fori_loop(0, kd // 16, fill_one, 0)

  pltpu.sync_copy(zero_v, deg_sh.at[pl.ds(s * zt, zt)])
  pltpu.sync_copy(dst_hbm.at[wid], dst_v)
  plsc.subcore_barrier()

  def body(j, _):
    pltpu.sync_copy(ones_v, deg_sh.at[dst_v.at[j]], add=True)
    return 0
  lax.fori_loop(0, nchunk, body, 0)

  plsc.subcore_barrier()
  pltpu.sync_copy(deg_sh.at[pl.ds(s * zt, zt)],
                  out_hbm.at[pl.ds(c * np_ + s * zt, zt)])


def _edge_body(hs_hbm, src_hbm, dst_hbm, out_hbm, acc_sh, src_v, dst_v,
               rows_v, sem, nchunk):
  c = lax.axis_index("c")
  s = lax.axis_index("s")
  wid = c * NS + s
  n = acc_sh.shape[0]
  rt = (n // NS + 7) // 8 * 8
  rt_last = n - (NS - 1) * rt
  ew = nchunk * K
  base = wid * ew

  @pl.when(s < NS - 1)
  def _():
    pltpu.sync_copy(hs_hbm.at[pl.ds(s * rt, rt)], acc_sh.at[pl.ds(s * rt, rt)])

  @pl.when(s == NS - 1)
  def _():
    pltpu.sync_copy(hs_hbm.at[pl.ds((NS - 1) * rt, rt_last)],
                    acc_sh.at[pl.ds((NS - 1) * rt, rt_last)])

  pltpu.sync_copy(src_hbm.at[pl.ds(base, ew)], src_v)
  pltpu.sync_copy(dst_hbm.at[wid], dst_v)
  plsc.subcore_barrier()

  sem_g, sem_s = sem
  pltpu.async_copy(hs_hbm.at[src_v.at[pl.ds(0, K)]], rows_v.at[0], sem_g)

  def body(j, _):
    b = lax.rem(j, 2)
    nb = 1 - b
    pltpu.make_async_copy(hs_hbm.at[src_v.at[pl.ds(j * K, K)]],
                          rows_v.at[b], sem_g).wait()

    @pl.when(j >= 1)
    def _():
      pltpu.make_async_copy(rows_v.at[nb], acc_sh.at[dst_v.at[j - 1]],
                            sem_s).wait()

    @pl.when(j + 1 < nchunk)
    def _():
      pltpu.async_copy(hs_hbm.at[src_v.at[pl.ds((j + 1) * K, K)]],
                       rows_v.at[nb], sem_g)

    pltpu.async_copy(rows_v.at[b], acc_sh.at[dst_v.at[j]], sem_s, add=True)
    return 0
  lax.fori_loop(0, nchunk, body, 0)

  lb = (nchunk - 1) % 2
  pltpu.make_async_copy(rows_v.at[lb], acc_sh.at[dst_v.at[nchunk - 1]],
                        sem_s).wait()
  plsc.subcore_barrier()

  @pl.when(s < NS - 1)
  def _():
    pltpu.sync_copy(acc_sh.at[pl.ds(s * rt, rt)],
                    out_hbm.at[pl.ds(c * n + s * rt, rt)])

  @pl.when(s == NS - 1)
  def _():
    pltpu.sync_copy(acc_sh.at[pl.ds((NS - 1) * rt, rt_last)],
                    out_hbm.at[pl.ds(c * n + (NS - 1) * rt, rt_last)])


def _first_tc(x_ref, w_ref, degt_ref, hs_ref, dis_ref):
  d = jnp.sum(degt_ref[...], axis=1, keepdims=True) + 1.0
  dis = lax.rsqrt(d)
  xw = jnp.dot(x_ref[...], w_ref[...], preferred_element_type=jnp.float32)
  hs_ref[...] = xw * dis
  dis_ref[...] = dis


def _mid_tc(acc0_ref, acc1_ref, hs_ref, dis_ref, b_ref, w_ref, out_ref):
  dis = dis_ref[...]
  h = (acc0_ref[...] + acc1_ref[...] - hs_ref[...]) * dis + b_ref[...]
  h = jnp.maximum(h, 0.0)
  out_ref[...] = jnp.dot(h, w_ref[...],
                         preferred_element_type=jnp.float32) * dis


def _head_tc(acc0_ref, acc1_ref, hs_ref, dis_ref, b_ref, wfc_ref, bfc_ref,
             out_ref, colsum):
  i = pl.program_id(0)
  nblk = pl.num_programs(0)
  dis = dis_ref[...]
  h = (acc0_ref[...] + acc1_ref[...] - hs_ref[...]) * dis + b_ref[...]
  h = jnp.maximum(h, 0.0)

  @pl.when(i == 0)
  def _():
    colsum[...] = jnp.zeros_like(colsum)

  colsum[...] += jnp.sum(h, axis=0, keepdims=True)

  @pl.when(i == nblk - 1)
  def _():
    g = colsum[...] / (nblk * h.shape[0])
    out_ref[...] = jnp.dot(g, wfc_ref[...],
                           preferred_element_type=jnp.float32) + bfc_ref[...]


def kernel(x, edge_index, W1, b1, W2, b2, W3, b3, Wfc, bfc):
  n, d = x.shape
  h = W1.shape[1]
  o = Wfc.shape[1]
  e = edge_index.shape[1]
  ew = e // NW
  nchunk = ew // K
  np_ = ((n + 255) // 256) * 256
  nblk = n // R

  src_flat = edge_index[0]
  dst3 = edge_index[1].reshape(NW, nchunk, K)
  kd = 80
  dst3b = edge_index[1].reshape(NW, ew // kd, kd)

  mesh = plsc.VectorSubcoreMesh(core_axis_name="c", subcore_axis_name="s")

  deg_kernel = pl.kernel(
      _deg_body,
      out_type=jax.ShapeDtypeStruct((NC * np_,), jnp.float32),
      mesh=mesh,
      scratch_types=[
          pltpu.VMEM_SHARED((np_,), jnp.float32),
          pltpu.VMEM((ew // kd, kd), jnp.int32),
          pltpu.VMEM((kd,), jnp.float32),
          pltpu.VMEM((np_ // NS,), jnp.float32),
      ],
  )
  deg_parts = deg_kernel(dst3b)
  degt = deg_parts.reshape(NC, np_).T[:n]

  edge_kernel = pl.kernel(
      functools.partial(_edge_body, nchunk=nchunk),
      out_type=jax.ShapeDtypeStruct((2 * n, h), jnp.float32),
      mesh=mesh,
      scratch_types=[
          pltpu.VMEM_SHARED((n, h), jnp.float32),
          pltpu.VMEM((ew,), jnp.int32),
          pltpu.VMEM((nchunk, K), jnp.int32),
          pltpu.VMEM((2, K, h), jnp.float32),
          (pltpu.SemaphoreType.DMA, pltpu.SemaphoreType.DMA),
      ],
  )

  row = lambda i: (i, 0)
  row_hi = lambda i: (i + nblk, 0)
  fixed = lambda i: (0, 0)

  first = pl.pallas_call(
      _first_tc,
      grid=(nblk,),
      in_specs=[
          pl.BlockSpec((R, d), row),
          pl.BlockSpec((d, h), fixed),
          pl.BlockSpec((R, NC), row),
      ],
      out_specs=[
          pl.BlockSpec((R, h), row),
          pl.BlockSpec((R, 1), row),
      ],
      out_shape=[
          jax.ShapeDtypeStruct((n, h), jnp.float32),
          jax.ShapeDtypeStruct((n, 1), jnp.float32),
      ],
  )
  hs1, dis = first(x, W1, degt)

  def mid(acc, hs_prev, b, w):
    return pl.pallas_call(
        _mid_tc,
        grid=(nblk,),
        in_specs=[
            pl.BlockSpec((R, h), row),
            pl.BlockSpec((R, h), row_hi),
            pl.BlockSpec((R, h), row),
            pl.BlockSpec((R, 1), row),
            pl.BlockSpec((1, h), fixed),
            pl.BlockSpec((h, h), fixed),
        ],
        out_specs=pl.BlockSpec((R, h), row),
        out_shape=jax.ShapeDtypeStruct((n, h), jnp.float32),
    )(acc, acc, hs_prev, dis, b.reshape(1, h), w)

  acc1 = edge_kernel(hs1, src_flat, dst3)
  hs2 = mid(acc1, hs1, b1, W2)
  acc2 = edge_kernel(hs2, src_flat, dst3)
  hs3 = mid(acc2, hs2, b2, W3)
  acc3 = edge_kernel(hs3, src_flat, dst3)

  wfc_p = jnp.zeros((h, 128), jnp.float32).at[:, :o].set(Wfc)
  bfc_p = jnp.zeros((1, 128), jnp.float32).at[0, :o].set(bfc)

  head = pl.pallas_call(
      _head_tc,
      grid=(nblk,),
      in_specs=[
          pl.BlockSpec((R, h), row),
          pl.BlockSpec((R, h), row_hi),
          pl.BlockSpec((R, h), row),
          pl.BlockSpec((R, 1), row),
          pl.BlockSpec((1, h), fixed),
          pl.BlockSpec((h, 128), fixed),
          pl.BlockSpec((1, 128), fixed),
      ],
      out_specs=pl.BlockSpec((1, 128), fixed),
      out_shape=jax.ShapeDtypeStruct((1, 128), jnp.float32),
      scratch_shapes=[pltpu.VMEM((1, 128), jnp.float32)],
  )
  out = head(acc3, acc3, hs3, dis, b3.reshape(1, h), wfc_p, bfc_p)
  return out[0, :o]


# speedup vs baseline: 2.1252x; 1.3813x over previous
import functools

import jax
import jax.numpy as jnp
from jax import lax
from jax.experimental import pallas as pl
from jax.experimental.pallas import tpu as pltpu
from jax.experimental.pallas import tpu_sc as plsc

NC = 2
NS = 16
NW = NC * NS
K = 80
NBUF = 3
R = 1000


def _deg_body(dst_hbm, out_hbm, deg_sh, dst_v, ones_v, zero_v):
  c = lax.axis_index("c")
  s = lax.axis_index("s")
  wid = c * NS + s
  np_, = deg_sh.shape
  zt = np_ // NS
  nchunk, kd = dst_v.shape

  def fill_zero(j, _):
    zero_v[pl.ds(j * 16, 16)] = jnp.zeros((16,), jnp.float32)
    return 0
  lax.fori_loop(0, zt // 16, fill_zero, 0)

  def fill_one(j, _):
    ones_v[pl.ds(j * 16, 16)] = jnp.ones((16,), jnp.float32)
    return 0
  lax.fori_loop(0, kd // 16, fill_one, 0)

  pltpu.sync_copy(zero_v, deg_sh.at[pl.ds(s * zt, zt)])
  pltpu.sync_copy(dst_hbm.at[wid], dst_v)
  plsc.subcore_barrier()

  def body(j, _):
    pltpu.sync_copy(ones_v, deg_sh.at[dst_v.at[j]], add=True)
    return 0
  lax.fori_loop(0, nchunk, body, 0)

  plsc.subcore_barrier()
  pltpu.sync_copy(deg_sh.at[pl.ds(s * zt, zt)],
                  out_hbm.at[pl.ds(c * np_ + s * zt, zt)])


def _edge_body(hs_hbm, src_hbm, dst_hbm, out_hbm, acc_sh, src_v, dst_v,
               rows_v, sem, nchunk):
  c = lax.axis_index("c")
  s = lax.axis_index("s")
  wid = c * NS + s
  n = acc_sh.shape[0]
  rt = (n // NS + 7) // 8 * 8
  rt_last = n - (NS - 1) * rt
  ew = nchunk * K
  base = wid * ew

  @pl.when(s < NS - 1)
  def _():
    pltpu.sync_copy(hs_hbm.at[pl.ds(s * rt, rt)], acc_sh.at[pl.ds(s * rt, rt)])

  @pl.when(s == NS - 1)
  def _():
    pltpu.sync_copy(hs_hbm.at[pl.ds((NS - 1) * rt, rt_last)],
                    acc_sh.at[pl.ds((NS - 1) * rt, rt_last)])

  pltpu.sync_copy(src_hbm.at[pl.ds(base, ew)], src_v)
  pltpu.sync_copy(dst_hbm.at[wid], dst_v)
  plsc.subcore_barrier()

  sem_g, sem_s = sem
  for p in range(NBUF - 1):
    pltpu.async_copy(hs_hbm.at[src_v.at[pl.ds(p * K, K)]], rows_v.at[p],
                     sem_g)

  def body(j, _):
    b = lax.rem(j, NBUF)
    pltpu.make_async_copy(hs_hbm.at[src_v.at[pl.ds(j * K, K)]],
                          rows_v.at[b], sem_g).wait()

    @pl.when(j >= NBUF - 1)
    def _():
      pltpu.make_async_copy(rows_v.at[lax.rem(j + 1, NBUF)],
                            acc_sh.at[dst_v.at[j - (NBUF - 1)]],
                            sem_s).wait()

    @pl.when(j + NBUF - 1 < nchunk)
    def _():
      pltpu.async_copy(hs_hbm.at[src_v.at[pl.ds((j + NBUF - 1) * K, K)]],
                       rows_v.at[lax.rem(j + NBUF - 1, NBUF)], sem_g)

    pltpu.async_copy(rows_v.at[b], acc_sh.at[dst_v.at[j]], sem_s, add=True)
    return 0
  lax.fori_loop(0, nchunk, body, 0)

  for p in range(NBUF - 1):
    j = nchunk - (NBUF - 1) + p
    pltpu.make_async_copy(rows_v.at[j % NBUF], acc_sh.at[dst_v.at[j]],
                          sem_s).wait()
  plsc.subcore_barrier()

  @pl.when(s < NS - 1)
  def _():
    pltpu.sync_copy(acc_sh.at[pl.ds(s * rt, rt)],
                    out_hbm.at[pl.ds(c * n + s * rt, rt)])

  @pl.when(s == NS - 1)
  def _():
    pltpu.sync_copy(acc_sh.at[pl.ds((NS - 1) * rt, rt_last)],
                    out_hbm.at[pl.ds(c * n + (NS - 1) * rt, rt_last)])


def _first_tc(x_ref, w_ref, degt_ref, hs_ref, dis_ref):
  d = jnp.sum(degt_ref[...], axis=1, keepdims=True) + 1.0
  dis = lax.rsqrt(d)
  xw = jnp.dot(x_ref[...], w_ref[...], preferred_element_type=jnp.float32)
  hs_ref[...] = xw * dis
  dis_ref[...] = dis


def _mid_tc(acc0_ref, acc1_ref, hs_ref, dis_ref, b_ref, w_ref, out_ref):
  dis = dis_ref[...]
  h = (acc0_ref[...] + acc1_ref[...] - hs_ref[...]) * dis + b_ref[...]
  h = jnp.maximum(h, 0.0)
  out_ref[...] = jnp.dot(h, w_ref[...],
                         preferred_element_type=jnp.float32) * dis


def _head_tc(acc0_ref, acc1_ref, hs_ref, dis_ref, b_ref, wfc_ref, bfc_ref,
             out_ref, colsum):
  i = pl.program_id(0)
  nblk = pl.num_programs(0)
  dis = dis_ref[...]
  h = (acc0_ref[...] + acc1_ref[...] - hs_ref[...]) * dis + b_ref[...]
  h = jnp.maximum(h, 0.0)

  @pl.when(i == 0)
  def _():
    colsum[...] = jnp.zeros_like(colsum)

  colsum[...] += jnp.sum(h, axis=0, keepdims=True)

  @pl.when(i == nblk - 1)
  def _():
    g = colsum[...] / (nblk * h.shape[0])
    out_ref[...] = jnp.dot(g, wfc_ref[...],
                           preferred_element_type=jnp.float32) + bfc_ref[...]


def kernel(x, edge_index, W1, b1, W2, b2, W3, b3, Wfc, bfc):
  n, d = x.shape
  h = W1.shape[1]
  o = Wfc.shape[1]
  e = edge_index.shape[1]
  ew = e // NW
  nchunk = ew // K
  np_ = ((n + 255) // 256) * 256
  nblk = n // R

  src_flat = edge_index[0]
  dst3 = edge_index[1].reshape(NW, nchunk, K)
  kd = 80
  dst3b = edge_index[1].reshape(NW, ew // kd, kd)

  mesh = plsc.VectorSubcoreMesh(core_axis_name="c", subcore_axis_name="s")

  deg_kernel = pl.kernel(
      _deg_body,
      out_type=jax.ShapeDtypeStruct((NC * np_,), jnp.float32),
      mesh=mesh,
      scratch_types=[
          pltpu.VMEM_SHARED((np_,), jnp.float32),
          pltpu.VMEM((ew // kd, kd), jnp.int32),
          pltpu.VMEM((kd,), jnp.float32),
          pltpu.VMEM((np_ // NS,), jnp.float32),
      ],
  )
  deg_parts = deg_kernel(dst3b)
  degt = deg_parts.reshape(NC, np_).T[:n]

  edge_kernel = pl.kernel(
      functools.partial(_edge_body, nchunk=nchunk),
      out_type=jax.ShapeDtypeStruct((2 * n, h), jnp.float32),
      mesh=mesh,
      compiler_params=pltpu.CompilerParams(use_tc_tiling_on_sc=False),
      scratch_types=[
          pltpu.VMEM_SHARED((n, h), jnp.float32),
          pltpu.VMEM((ew,), jnp.int32),
          pltpu.VMEM((nchunk, K), jnp.int32),
          pltpu.VMEM((NBUF, K, h), jnp.float32),
          (pltpu.SemaphoreType.DMA, pltpu.SemaphoreType.DMA),
      ],
  )

  row = lambda i: (i, 0)
  row_hi = lambda i: (i + nblk, 0)
  fixed = lambda i: (0, 0)

  first = pl.pallas_call(
      _first_tc,
      grid=(nblk,),
      in_specs=[
          pl.BlockSpec((R, d), row),
          pl.BlockSpec((d, h), fixed),
          pl.BlockSpec((R, NC), row),
      ],
      out_specs=[
          pl.BlockSpec((R, h), row),
          pl.BlockSpec((R, 1), row),
      ],
      out_shape=[
          jax.ShapeDtypeStruct((n, h), jnp.float32),
          jax.ShapeDtypeStruct((n, 1), jnp.float32),
      ],
  )
  hs1, dis = first(x, W1, degt)

  def mid(acc, hs_prev, b, w):
    return pl.pallas_call(
        _mid_tc,
        grid=(nblk,),
        in_specs=[
            pl.BlockSpec((R, h), row),
            pl.BlockSpec((R, h), row_hi),
            pl.BlockSpec((R, h), row),
            pl.BlockSpec((R, 1), row),
            pl.BlockSpec((1, h), fixed),
            pl.BlockSpec((h, h), fixed),
        ],
        out_specs=pl.BlockSpec((R, h), row),
        out_shape=jax.ShapeDtypeStruct((n, h), jnp.float32),
    )(acc, acc, hs_prev, dis, b.reshape(1, h), w)

  acc1 = edge_kernel(hs1, src_flat, dst3)
  hs2 = mid(acc1, hs1, b1, W2)
  acc2 = edge_kernel(hs2, src_flat, dst3)
  hs3 = mid(acc2, hs2, b2, W3)
  acc3 = edge_kernel(hs3, src_flat, dst3)

  wfc_p = jnp.zeros((h, 128), jnp.float32).at[:, :o].set(Wfc)
  bfc_p = jnp.zeros((1, 128), jnp.float32).at[0, :o].set(bfc)

  head = pl.pallas_call(
      _head_tc,
      grid=(nblk,),
      in_specs=[
          pl.BlockSpec((R, h), row),
          pl.BlockSpec((R, h), row_hi),
          pl.BlockSpec((R, h), row),
          pl.BlockSpec((R, 1), row),
          pl.BlockSpec((1, h), fixed),
          pl.BlockSpec((h, 128), fixed),
          pl.BlockSpec((1, 128), fixed),
      ],
      out_specs=pl.BlockSpec((1, 128), fixed),
      out_shape=jax.ShapeDtypeStruct((1, 128), jnp.float32),
      scratch_shapes=[pltpu.VMEM((1, 128), jnp.float32)],
  )
  out = head(acc3, acc3, hs3, dis, b3.reshape(1, h), wfc_p, bfc_p)
  return out[0, :o]


# speedup vs baseline: 2.1799x; 1.0257x over previous
import functools

import jax
import jax.numpy as jnp
from jax import lax
from jax.experimental import pallas as pl
from jax.experimental.pallas import tpu as pltpu
from jax.experimental.pallas import tpu_sc as plsc

NC = 2
NS = 16
NW = NC * NS
K = 80
NBUF = 3
R = 2000


def _deg_body(dst_hbm, out_hbm, deg_sh, dst_v, ones_v, zero_v):
  c = lax.axis_index("c")
  s = lax.axis_index("s")
  wid = c * NS + s
  np_, = deg_sh.shape
  zt = np_ // NS
  nchunk, kd = dst_v.shape

  def fill_zero(j, _):
    zero_v[pl.ds(j * 16, 16)] = jnp.zeros((16,), jnp.float32)
    return 0
  lax.fori_loop(0, zt // 16, fill_zero, 0)

  def fill_one(j, _):
    ones_v[pl.ds(j * 16, 16)] = jnp.ones((16,), jnp.float32)
    return 0
  lax.fori_loop(0, kd // 16, fill_one, 0)

  pltpu.sync_copy(zero_v, deg_sh.at[pl.ds(s * zt, zt)])
  pltpu.sync_copy(dst_hbm.at[wid], dst_v)
  plsc.subcore_barrier()

  def body(j, _):
    pltpu.sync_copy(ones_v, deg_sh.at[dst_v.at[j]], add=True)
    return 0
  lax.fori_loop(0, nchunk, body, 0)

  plsc.subcore_barrier()
  pltpu.sync_copy(deg_sh.at[pl.ds(s * zt, zt)],
                  out_hbm.at[pl.ds(c * np_ + s * zt, zt)])


def _edge_body(hs_hbm, zeros_hbm, src_hbm, dst_hbm, out_hbm, acc_sh, src_v,
               dst_v, rows_v, sem, nchunk):
  c = lax.axis_index("c")
  s = lax.axis_index("s")
  wid = c * NS + s
  n = acc_sh.shape[0]
  rt = (n // NS + 7) // 8 * 8
  rt_last = n - (NS - 1) * rt
  ew = nchunk * K
  base = wid * ew

  @pl.when((c == 0) & (s < NS - 1))
  def _():
    pltpu.sync_copy(hs_hbm.at[pl.ds(s * rt, rt)], acc_sh.at[pl.ds(s * rt, rt)])

  @pl.when((c == 0) & (s == NS - 1))
  def _():
    pltpu.sync_copy(hs_hbm.at[pl.ds((NS - 1) * rt, rt_last)],
                    acc_sh.at[pl.ds((NS - 1) * rt, rt_last)])

  @pl.when((c == 1) & (s < NS - 1))
  def _():
    pltpu.sync_copy(zeros_hbm, acc_sh.at[pl.ds(s * rt, rt)])

  @pl.when((c == 1) & (s == NS - 1))
  def _():
    pltpu.sync_copy(zeros_hbm.at[pl.ds(0, rt_last)],
                    acc_sh.at[pl.ds((NS - 1) * rt, rt_last)])

  pltpu.sync_copy(src_hbm.at[pl.ds(base, ew)], src_v)
  pltpu.sync_copy(dst_hbm.at[wid], dst_v)
  plsc.subcore_barrier()

  sem_g, sem_s = sem
  for p in range(NBUF - 1):
    pltpu.async_copy(hs_hbm.at[src_v.at[pl.ds(p * K, K)]], rows_v.at[p],
                     sem_g)

  def body(j, _):
    b = lax.rem(j, NBUF)
    pltpu.make_async_copy(hs_hbm.at[src_v.at[pl.ds(j * K, K)]],
                          rows_v.at[b], sem_g).wait()

    @pl.when(j >= NBUF - 1)
    def _():
      pltpu.make_async_copy(rows_v.at[lax.rem(j + 1, NBUF)],
                            acc_sh.at[dst_v.at[j - (NBUF - 1)]],
                            sem_s).wait()

    @pl.when(j + NBUF - 1 < nchunk)
    def _():
      pltpu.async_copy(hs_hbm.at[src_v.at[pl.ds((j + NBUF - 1) * K, K)]],
                       rows_v.at[lax.rem(j + NBUF - 1, NBUF)], sem_g)

    pltpu.async_copy(rows_v.at[b], acc_sh.at[dst_v.at[j]], sem_s, add=True)
    return 0
  lax.fori_loop(0, nchunk, body, 0)

  for p in range(NBUF - 1):
    j = nchunk - (NBUF - 1) + p
    pltpu.make_async_copy(rows_v.at[j % NBUF], acc_sh.at[dst_v.at[j]],
                          sem_s).wait()
  plsc.subcore_barrier()

  @pl.when(s < NS - 1)
  def _():
    pltpu.sync_copy(acc_sh.at[pl.ds(s * rt, rt)],
                    out_hbm.at[pl.ds(c * n + s * rt, rt)])

  @pl.when(s == NS - 1)
  def _():
    pltpu.sync_copy(acc_sh.at[pl.ds((NS - 1) * rt, rt_last)],
                    out_hbm.at[pl.ds(c * n + (NS - 1) * rt, rt_last)])


def _first_tc(x_ref, w_ref, degt_ref, hs_ref, dis_ref):
  d = jnp.sum(degt_ref[...], axis=1, keepdims=True) + 1.0
  dis = lax.rsqrt(d)
  xw = jnp.dot(x_ref[...], w_ref[...], preferred_element_type=jnp.float32)
  hs_ref[...] = xw * dis
  dis_ref[...] = dis


def _mid_tc(acc0_ref, acc1_ref, dis_ref, b_ref, w_ref, out_ref):
  dis = dis_ref[...]
  h = (acc0_ref[...] + acc1_ref[...]) * dis + b_ref[...]
  h = jnp.maximum(h, 0.0)
  out_ref[...] = jnp.dot(h, w_ref[...],
                         preferred_element_type=jnp.float32) * dis


def _head_tc(acc0_ref, acc1_ref, dis_ref, b_ref, wfc_ref, bfc_ref,
             out_ref, colsum):
  i = pl.program_id(0)
  nblk = pl.num_programs(0)
  dis = dis_ref[...]
  h = (acc0_ref[...] + acc1_ref[...]) * dis + b_ref[...]
  h = jnp.maximum(h, 0.0)

  @pl.when(i == 0)
  def _():
    colsum[...] = jnp.zeros_like(colsum)

  colsum[...] += jnp.sum(h, axis=0, keepdims=True)

  @pl.when(i == nblk - 1)
  def _():
    g = colsum[...] / (nblk * h.shape[0])
    out_ref[...] = jnp.dot(g, wfc_ref[...],
                           preferred_element_type=jnp.float32) + bfc_ref[...]


def kernel(x, edge_index, W1, b1, W2, b2, W3, b3, Wfc, bfc):
  n, d = x.shape
  h = W1.shape[1]
  o = Wfc.shape[1]
  e = edge_index.shape[1]
  ew = e // NW
  nchunk = ew // K
  np_ = ((n + 255) // 256) * 256
  nblk = n // R

  src_flat = edge_index[0]
  dst3 = edge_index[1].reshape(NW, nchunk, K)
  kd = 80
  dst3b = edge_index[1].reshape(NW, ew // kd, kd)

  mesh = plsc.VectorSubcoreMesh(core_axis_name="c", subcore_axis_name="s")

  deg_kernel = pl.kernel(
      _deg_body,
      out_type=jax.ShapeDtypeStruct((NC * np_,), jnp.float32),
      mesh=mesh,
      scratch_types=[
          pltpu.VMEM_SHARED((np_,), jnp.float32),
          pltpu.VMEM((ew // kd, kd), jnp.int32),
          pltpu.VMEM((kd,), jnp.float32),
          pltpu.VMEM((np_ // NS,), jnp.float32),
      ],
  )
  deg_parts = deg_kernel(dst3b)
  degt = deg_parts.reshape(NC, np_).T[:n]

  rt = (n // NS + 7) // 8 * 8
  zeros_rt = jnp.zeros((rt, h), jnp.float32)

  edge_kernel = pl.kernel(
      functools.partial(_edge_body, nchunk=nchunk),
      out_type=jax.ShapeDtypeStruct((2 * n, h), jnp.float32),
      mesh=mesh,
      compiler_params=pltpu.CompilerParams(use_tc_tiling_on_sc=False),
      scratch_types=[
          pltpu.VMEM_SHARED((n, h), jnp.float32),
          pltpu.VMEM((ew,), jnp.int32),
          pltpu.VMEM((nchunk, K), jnp.int32),
          pltpu.VMEM((NBUF, K, h), jnp.float32),
          (pltpu.SemaphoreType.DMA, pltpu.SemaphoreType.DMA),
      ],
  )

  row = lambda i: (i, 0)
  row_hi = lambda i: (i + nblk, 0)
  fixed = lambda i: (0, 0)

  first = pl.pallas_call(
      _first_tc,
      grid=(nblk,),
      in_specs=[
          pl.BlockSpec((R, d), row),
          pl.BlockSpec((d, h), fixed),
          pl.BlockSpec((R, NC), row),
      ],
      out_specs=[
          pl.BlockSpec((R, h), row),
          pl.BlockSpec((R, 1), row),
      ],
      out_shape=[
          jax.ShapeDtypeStruct((n, h), jnp.float32),
          jax.ShapeDtypeStruct((n, 1), jnp.float32),
      ],
  )
  hs1, dis = first(x, W1, degt)

  def mid(acc, b, w):
    return pl.pallas_call(
        _mid_tc,
        grid=(nblk,),
        in_specs=[
            pl.BlockSpec((R, h), row),
            pl.BlockSpec((R, h), row_hi),
            pl.BlockSpec((R, 1), row),
            pl.BlockSpec((1, h), fixed),
            pl.BlockSpec((h, h), fixed),
        ],
        out_specs=pl.BlockSpec((R, h), row),
        out_shape=jax.ShapeDtypeStruct((n, h), jnp.float32),
    )(acc, acc, dis, b.reshape(1, h), w)

  acc1 = edge_kernel(hs1, zeros_rt, src_flat, dst3)
  hs2 = mid(acc1, b1, W2)
  acc2 = edge_kernel(hs2, zeros_rt, src_flat, dst3)
  hs3 = mid(acc2, b2, W3)
  acc3 = edge_kernel(hs3, zeros_rt, src_flat, dst3)

  wfc_p = jnp.zeros((h, 128), jnp.float32).at[:, :o].set(Wfc)
  bfc_p = jnp.zeros((1, 128), jnp.float32).at[0, :o].set(bfc)

  head = pl.pallas_call(
      _head_tc,
      grid=(nblk,),
      in_specs=[
          pl.BlockSpec((R, h), row),
          pl.BlockSpec((R, h), row_hi),
          pl.BlockSpec((R, 1), row),
          pl.BlockSpec((1, h), fixed),
          pl.BlockSpec((h, 128), fixed),
          pl.BlockSpec((1, 128), fixed),
      ],
      out_specs=pl.BlockSpec((1, 128), fixed),
      out_shape=jax.ShapeDtypeStruct((1, 128), jnp.float32),
      scratch_shapes=[pltpu.VMEM((1, 128), jnp.float32)],
  )
  out = head(acc3, acc3, dis, b3.reshape(1, h), wfc_p, bfc_p)
  return out[0, :o]


# speedup vs baseline: 2.2213x; 1.0190x over previous
import functools

import jax
import jax.numpy as jnp
from jax import lax
from jax.experimental import pallas as pl
from jax.experimental.pallas import tpu as pltpu
from jax.experimental.pallas import tpu_sc as plsc

NC = 2
NS = 16
NW = NC * NS
K = 80
NBUF = 3
R = 2000


def _deg_body(dst_hbm, out_hbm, deg_sh, dst_v, ones_v, zero_v):
  c = lax.axis_index("c")
  s = lax.axis_index("s")
  wid = c * NS + s
  np_, = deg_sh.shape
  zt = np_ // NS
  nchunk, kd = dst_v.shape

  def fill_zero(j, _):
    zero_v[pl.ds(j * 16, 16)] = jnp.zeros((16,), jnp.float32)
    return 0
  lax.fori_loop(0, zt // 16, fill_zero, 0)

  def fill_one(j, _):
    ones_v[pl.ds(j * 16, 16)] = jnp.ones((16,), jnp.float32)
    return 0
  lax.fori_loop(0, kd // 16, fill_one, 0)

  pltpu.sync_copy(zero_v, deg_sh.at[pl.ds(s * zt, zt)])
  pltpu.sync_copy(dst_hbm.at[wid], dst_v)
  plsc.subcore_barrier()

  def body(j, _):
    pltpu.sync_copy(ones_v, deg_sh.at[dst_v.at[j]], add=True)
    return 0
  lax.fori_loop(0, nchunk, body, 0)

  plsc.subcore_barrier()
  pltpu.sync_copy(deg_sh.at[pl.ds(s * zt, zt)],
                  out_hbm.at[pl.ds(c * np_ + s * zt, zt)])


def _edge_body(hs_hbm, zeros_hbm, src_hbm, dst_hbm, out_hbm, acc_sh, src_v,
               dst_v, rows_v, sem, nchunk):
  c = lax.axis_index("c")
  s = lax.axis_index("s")
  wid = c * NS + s
  n = acc_sh.shape[0]
  rt = (n // NS + 7) // 8 * 8
  rt_last = n - (NS - 1) * rt
  ew = nchunk * K
  base = wid * ew

  sem_g, sem_s = sem
  pltpu.async_copy(src_hbm.at[pl.ds(base, ew)], src_v, sem_g)
  pltpu.async_copy(dst_hbm.at[wid], dst_v, sem_s)

  @pl.when((c == 0) & (s < NS - 1))
  def _():
    pltpu.sync_copy(hs_hbm.at[pl.ds(s * rt, rt)], acc_sh.at[pl.ds(s * rt, rt)])

  @pl.when((c == 0) & (s == NS - 1))
  def _():
    pltpu.sync_copy(hs_hbm.at[pl.ds((NS - 1) * rt, rt_last)],
                    acc_sh.at[pl.ds((NS - 1) * rt, rt_last)])

  @pl.when((c == 1) & (s < NS - 1))
  def _():
    pltpu.sync_copy(zeros_hbm, acc_sh.at[pl.ds(s * rt, rt)])

  @pl.when((c == 1) & (s == NS - 1))
  def _():
    pltpu.sync_copy(zeros_hbm.at[pl.ds(0, rt_last)],
                    acc_sh.at[pl.ds((NS - 1) * rt, rt_last)])

  pltpu.make_async_copy(src_hbm.at[pl.ds(base, ew)], src_v, sem_g).wait()
  pltpu.make_async_copy(dst_hbm.at[wid], dst_v, sem_s).wait()
  plsc.subcore_barrier()
  for p in range(NBUF - 1):
    pltpu.async_copy(hs_hbm.at[src_v.at[pl.ds(p * K, K)]], rows_v.at[p],
                     sem_g)

  def body(j, _):
    b = lax.rem(j, NBUF)
    pltpu.make_async_copy(hs_hbm.at[src_v.at[pl.ds(j * K, K)]],
                          rows_v.at[b], sem_g).wait()

    @pl.when(j >= NBUF - 1)
    def _():
      pltpu.make_async_copy(rows_v.at[lax.rem(j + 1, NBUF)],
                            acc_sh.at[dst_v.at[j - (NBUF - 1)]],
                            sem_s).wait()

    @pl.when(j + NBUF - 1 < nchunk)
    def _():
      pltpu.async_copy(hs_hbm.at[src_v.at[pl.ds((j + NBUF - 1) * K, K)]],
                       rows_v.at[lax.rem(j + NBUF - 1, NBUF)], sem_g)

    pltpu.async_copy(rows_v.at[b], acc_sh.at[dst_v.at[j]], sem_s, add=True)
    return 0
  lax.fori_loop(0, nchunk, body, 0)

  for p in range(NBUF - 1):
    j = nchunk - (NBUF - 1) + p
    pltpu.make_async_copy(rows_v.at[j % NBUF], acc_sh.at[dst_v.at[j]],
                          sem_s).wait()
  plsc.subcore_barrier()

  @pl.when(s < NS - 1)
  def _():
    pltpu.sync_copy(acc_sh.at[pl.ds(s * rt, rt)],
                    out_hbm.at[pl.ds(c * n + s * rt, rt)])

  @pl.when(s == NS - 1)
  def _():
    pltpu.sync_copy(acc_sh.at[pl.ds((NS - 1) * rt, rt_last)],
                    out_hbm.at[pl.ds(c * n + (NS - 1) * rt, rt_last)])


def _mm_tc(x_ref, w_ref, out_ref):
  out_ref[...] = jnp.dot(x_ref[...], w_ref[...],
                         preferred_element_type=jnp.float32)


def _scale_tc(xw_ref, degt_ref, hs_ref, dis_ref):
  d = jnp.sum(degt_ref[...], axis=1, keepdims=True) + 1.0
  dis = lax.rsqrt(d)
  hs_ref[...] = xw_ref[...] * dis
  dis_ref[...] = dis


def _mid_tc(acc0_ref, acc1_ref, dis_ref, b_ref, w_ref, out_ref):
  dis = dis_ref[...]
  h = (acc0_ref[...] + acc1_ref[...]) * dis + b_ref[...]
  h = jnp.maximum(h, 0.0)
  out_ref[...] = jnp.dot(h, w_ref[...],
                         preferred_element_type=jnp.float32) * dis


def _head_tc(acc0_ref, acc1_ref, dis_ref, b_ref, wfc_ref, bfc_ref,
             out_ref, colsum):
  i = pl.program_id(0)
  nblk = pl.num_programs(0)
  dis = dis_ref[...]
  h = (acc0_ref[...] + acc1_ref[...]) * dis + b_ref[...]
  h = jnp.maximum(h, 0.0)

  @pl.when(i == 0)
  def _():
    colsum[...] = jnp.zeros_like(colsum)

  colsum[...] += jnp.sum(h, axis=0, keepdims=True)

  @pl.when(i == nblk - 1)
  def _():
    g = colsum[...] / (nblk * h.shape[0])
    out_ref[...] = jnp.dot(g, wfc_ref[...],
                           preferred_element_type=jnp.float32) + bfc_ref[...]


def kernel(x, edge_index, W1, b1, W2, b2, W3, b3, Wfc, bfc):
  n, d = x.shape
  h = W1.shape[1]
  o = Wfc.shape[1]
  e = edge_index.shape[1]
  ew = e // NW
  nchunk = ew // K
  np_ = ((n + 255) // 256) * 256
  nblk = n // R

  src_flat = edge_index[0]
  dst3 = edge_index[1].reshape(NW, nchunk, K)
  kd = 80
  dst3b = edge_index[1].reshape(NW, ew // kd, kd)

  mesh = plsc.VectorSubcoreMesh(core_axis_name="c", subcore_axis_name="s")

  deg_kernel = pl.kernel(
      _deg_body,
      out_type=jax.ShapeDtypeStruct((NC * np_,), jnp.float32),
      mesh=mesh,
      scratch_types=[
          pltpu.VMEM_SHARED((np_,), jnp.float32),
          pltpu.VMEM((ew // kd, kd), jnp.int32),
          pltpu.VMEM((kd,), jnp.float32),
          pltpu.VMEM((np_ // NS,), jnp.float32),
      ],
  )
  deg_parts = deg_kernel(dst3b)
  degt = deg_parts.reshape(NC, np_).T[:n]

  rt = (n // NS + 7) // 8 * 8
  zeros_rt = jnp.zeros((rt, h), jnp.float32)

  edge_kernel = pl.kernel(
      functools.partial(_edge_body, nchunk=nchunk),
      out_type=jax.ShapeDtypeStruct((2 * n, h), jnp.float32),
      mesh=mesh,
      compiler_params=pltpu.CompilerParams(use_tc_tiling_on_sc=False),
      scratch_types=[
          pltpu.VMEM_SHARED((n, h), jnp.float32),
          pltpu.VMEM((ew,), jnp.int32),
          pltpu.VMEM((nchunk, K), jnp.int32),
          pltpu.VMEM((NBUF, K, h), jnp.float32),
          (pltpu.SemaphoreType.DMA, pltpu.SemaphoreType.DMA),
      ],
  )

  row = lambda i: (i, 0)
  row_hi = lambda i: (i + nblk, 0)
  fixed = lambda i: (0, 0)

  xw1 = pl.pallas_call(
      _mm_tc,
      grid=(nblk,),
      in_specs=[
          pl.BlockSpec((R, d), row),
          pl.BlockSpec((d, h), fixed),
      ],
      out_specs=pl.BlockSpec((R, h), row),
      out_shape=jax.ShapeDtypeStruct((n, h), jnp.float32),
  )(x, W1)

  hs1, dis = pl.pallas_call(
      _scale_tc,
      grid=(nblk,),
      in_specs=[
          pl.BlockSpec((R, h), row),
          pl.BlockSpec((R, NC), row),
      ],
      out_specs=[
          pl.BlockSpec((R, h), row),
          pl.BlockSpec((R, 1), row),
      ],
      out_shape=[
          jax.ShapeDtypeStruct((n, h), jnp.float32),
          jax.ShapeDtypeStruct((n, 1), jnp.float32),
      ],
  )(xw1, degt)

  def mid(acc, b, w):
    return pl.pallas_call(
        _mid_tc,
        grid=(nblk,),
        in_specs=[
            pl.BlockSpec((R, h), row),
            pl.BlockSpec((R, h), row_hi),
            pl.BlockSpec((R, 1), row),
            pl.BlockSpec((1, h), fixed),
            pl.BlockSpec((h, h), fixed),
        ],
        out_specs=pl.BlockSpec((R, h), row),
        out_shape=jax.ShapeDtypeStruct((n, h), jnp.float32),
    )(acc, acc, dis, b.reshape(1, h), w)

  acc1 = edge_kernel(hs1, zeros_rt, src_flat, dst3)
  hs2 = mid(acc1, b1, W2)
  acc2 = edge_kernel(hs2, zeros_rt, src_flat, dst3)
  hs3 = mid(acc2, b2, W3)
  acc3 = edge_kernel(hs3, zeros_rt, src_flat, dst3)

  wfc_p = jnp.zeros((h, 128), jnp.float32).at[:, :o].set(Wfc)
  bfc_p = jnp.zeros((1, 128), jnp.float32).at[0, :o].set(bfc)

  head = pl.pallas_call(
      _head_tc,
      grid=(nblk,),
      in_specs=[
          pl.BlockSpec((R, h), row),
          pl.BlockSpec((R, h), row_hi),
          pl.BlockSpec((R, 1), row),
          pl.BlockSpec((1, h), fixed),
          pl.BlockSpec((h, 128), fixed),
          pl.BlockSpec((1, 128), fixed),
      ],
      out_specs=pl.BlockSpec((1, 128), fixed),
      out_shape=jax.ShapeDtypeStruct((1, 128), jnp.float32),
      scratch_shapes=[pltpu.VMEM((1, 128), jnp.float32)],
  )
  out = head(acc3, acc3, dis, b3.reshape(1, h), wfc_p, bfc_p)
  return out[0, :o]
